# Initial kernel scaffold; baseline (speedup 1.0000x reference)
import jax
import jax.numpy as jnp
from jax.experimental import pallas as pl


def kernel(inputs, pos_table):
    raise NotImplementedError("write your pallas kernel here")


# speedup vs baseline: 1.7597x; 1.7597x over previous
import jax
import jax.numpy as jnp
from jax.experimental import pallas as pl

_SEQ_BLOCK = 256


def _add_kernel(in_ref, pos_ref, out_ref):
    out_ref[...] = in_ref[...] + pos_ref[...][None, :, :]


def kernel(inputs, pos_table):
    batch, seq_len, dim = inputs.shape
    s_blk = _SEQ_BLOCK if seq_len % _SEQ_BLOCK == 0 else seq_len
    grid = (seq_len // s_blk,)
    return pl.pallas_call(
        _add_kernel,
        grid=grid,
        in_specs=[
            pl.BlockSpec((batch, s_blk, dim), lambda i: (0, i, 0)),
            pl.BlockSpec((s_blk, dim), lambda i: (i, 0)),
        ],
        out_specs=pl.BlockSpec((batch, s_blk, dim), lambda i: (0, i, 0)),
        out_shape=jax.ShapeDtypeStruct(inputs.shape, inputs.dtype),
    )(inputs, pos_table)


# speedup vs baseline: 1.8058x; 1.0262x over previous
import jax
import jax.numpy as jnp
from jax.experimental import pallas as pl

_SEQ_BLOCK = 512


def _add_kernel(in_ref, pos_ref, out_ref):
    out_ref[...] = in_ref[...] + pos_ref[...][None, :, :]


def kernel(inputs, pos_table):
    batch, seq_len, dim = inputs.shape
    s_blk = _SEQ_BLOCK if seq_len % _SEQ_BLOCK == 0 else seq_len
    grid = (seq_len // s_blk,)
    return pl.pallas_call(
        _add_kernel,
        grid=grid,
        in_specs=[
            pl.BlockSpec((batch, s_blk, dim), lambda i: (0, i, 0)),
            pl.BlockSpec((s_blk, dim), lambda i: (i, 0)),
        ],
        out_specs=pl.BlockSpec((batch, s_blk, dim), lambda i: (0, i, 0)),
        out_shape=jax.ShapeDtypeStruct(inputs.shape, inputs.dtype),
    )(inputs, pos_table)


# speedup vs baseline: 1.8071x; 1.0007x over previous
import jax
import jax.numpy as jnp
from jax.experimental import pallas as pl

_SEQ_BLOCK = 1024


def _add_kernel(in_ref, pos_ref, out_ref):
    out_ref[...] = in_ref[...] + pos_ref[...][None, :, :]


def kernel(inputs, pos_table):
    batch, seq_len, dim = inputs.shape
    s_blk = _SEQ_BLOCK if seq_len % _SEQ_BLOCK == 0 else seq_len
    grid = (seq_len // s_blk,)
    return pl.pallas_call(
        _add_kernel,
        grid=grid,
        in_specs=[
            pl.BlockSpec((batch, s_blk, dim), lambda i: (0, i, 0)),
            pl.BlockSpec((s_blk, dim), lambda i: (i, 0)),
        ],
        out_specs=pl.BlockSpec((batch, s_blk, dim), lambda i: (0, i, 0)),
        out_shape=jax.ShapeDtypeStruct(inputs.shape, inputs.dtype),
    )(inputs, pos_table)
